# baseline (speedup 1.0000x reference)
import functools

import jax
import jax.numpy as jnp
from jax import lax
from jax.experimental import pallas as pl
from jax.experimental.pallas import tpu as pltpu
from jax.experimental.pallas import tpu_sc as plsc

NC = 2
NS = 16
LANES = 16
BLKC = 512
NBLK = 196
LASTC0 = 99456
OROW = BLKC // 2
RND = 128
TAILC = 99968


def _transpose_block(src, dst, j, in_v, out_v, sem, lane):
    c0 = pl.multiple_of(jnp.minimum(j * BLKC, LASTC0), 128)
    copies = [
        pltpu.async_copy(src.at[pl.ds(8 * q, 8), pl.ds(c0, BLKC)],
                         in_v.at[pl.ds(8 * q, 8)], sem)
        for q in range(8)
    ]
    for c in copies:
        c.wait()

    def row_body(r, carry):
        for p in range(2):
            col = jnp.full((LANES,), 2 * r + p, jnp.int32)
            for h in range(4):
                rows = lane + h * LANES
                out_v[r, pl.ds(p * 64 + h * LANES, LANES)] = (
                    plsc.load_gather(in_v, [rows, col]))
        return carry

    lax.fori_loop(0, OROW, row_body, 0)
    r0 = pl.multiple_of(jnp.right_shift(c0, 1), 64)
    pltpu.sync_copy(out_v, dst.at[pl.ds(r0, OROW)])


def _t_kernel(u_t, m_t, utail, mtail, ut2, mt2, in_v, out_v, tail_v, sem):
    cid = lax.axis_index("c")
    sid = lax.axis_index("s")
    wid = sid * NC + cid
    lane = lax.iota(jnp.int32, LANES)
    for k in range(13):
        blk = wid + 32 * k

        @pl.when(blk < NBLK)
        def _u():
            _transpose_block(u_t, ut2, blk, in_v, out_v, sem, lane)

        @pl.when(jnp.logical_and(blk >= NBLK, blk < 2 * NBLK))
        def _m():
            _transpose_block(m_t, mt2, blk - NBLK, in_v, out_v, sem, lane)

    @pl.when(wid == 0)
    def _tails():
        pltpu.sync_copy(utail, tail_v)
        pltpu.sync_copy(tail_v, ut2.at[pl.ds(TAILC // 2, 16)])
        pltpu.sync_copy(mtail, tail_v)
        pltpu.sync_copy(tail_v, mt2.at[pl.ds(TAILC // 2, 16)])


def _g_round(t2, row_v, buf, r, sem):
    return pltpu.async_copy(t2.at[row_v.at[pl.ds(r * RND, RND)]], buf, sem)


def _g_compute(uidx_v, midx_v, ubuf, mbuf, o_v, r, lane):
    def group_body(g, carry):
        u16 = uidx_v[pl.ds(r * RND + g * LANES, LANES)]
        m16 = midx_v[pl.ds(r * RND + g * LANES, LANES)]
        uoff = (u16 & 1) * 64
        moff = (m16 & 1) * 64
        rows = lane + g * LANES
        acc = jnp.zeros((LANES,), jnp.float32)
        for dd in range(64):
            acc = acc + (plsc.load_gather(ubuf, [rows, uoff + dd])
                         * plsc.load_gather(mbuf, [rows, moff + dd]))
        o_v[pl.ds(r * RND + g * LANES, LANES)] = acc
        return carry

    lax.fori_loop(0, RND // LANES, group_body, 0)


def _d_kernel(bpw, ut2, mt2, uidx_hbm, midx_hbm, out_hbm,
              uidx_v, midx_v, urow_v, mrow_v, o_v,
              ub0, ub1, mb0, mb1, sem0, sem1):
    cid = lax.axis_index("c")
    sid = lax.axis_index("s")
    wid = sid * NC + cid
    base = wid * bpw
    lane = lax.iota(jnp.int32, LANES)
    nr = bpw // RND
    ubufs, mbufs, sems = (ub0, ub1), (mb0, mb1), (sem0, sem1)

    pltpu.sync_copy(uidx_hbm.at[pl.ds(base, bpw)], uidx_v)
    pltpu.sync_copy(midx_hbm.at[pl.ds(base, bpw)], midx_v)

    for g in range(bpw // LANES):
        sl = pl.ds(g * LANES, LANES)
        urow_v[sl] = jnp.right_shift(uidx_v[sl], 1)
        mrow_v[sl] = jnp.right_shift(midx_v[sl], 1)

    _g_round(ut2, urow_v, ubufs[0], 0, sems[0])
    _g_round(mt2, mrow_v, mbufs[0], 0, sems[0])
    _g_round(ut2, urow_v, ubufs[1], 1, sems[1])
    _g_round(mt2, mrow_v, mbufs[1], 1, sems[1])
    for r in range(nr):
        b = r % 2
        pltpu.make_async_copy(
            ut2.at[urow_v.at[pl.ds(r * RND, RND)]], ubufs[b], sems[b]).wait()
        pltpu.make_async_copy(
            mt2.at[mrow_v.at[pl.ds(r * RND, RND)]], mbufs[b], sems[b]).wait()
        _g_compute(uidx_v, midx_v, ubufs[b], mbufs[b], o_v, r, lane)
        if r + 2 < nr:
            _g_round(ut2, urow_v, ubufs[b], r + 2, sems[b])
            _g_round(mt2, mrow_v, mbufs[b], r + 2, sems[b])

    pltpu.sync_copy(o_v, out_hbm.at[pl.ds(base, bpw)])


def kernel(x, user_emb, movie_emb):
    b = x.shape[1]
    v = user_emb.shape[0]
    bpw = b // (NC * NS)

    u_t = user_emb.T
    m_t = movie_emb.T
    utail = user_emb[TAILC:].reshape(16, 128)
    mtail = movie_emb[TAILC:].reshape(16, 128)
    mesh = plsc.VectorSubcoreMesh(core_axis_name="c", subcore_axis_name="s")
    params = pltpu.CompilerParams(
        use_tc_tiling_on_sc=True, needs_layout_passes=False)

    t_run = functools.partial(
        pl.kernel, mesh=mesh,
        out_type=(jax.ShapeDtypeStruct((v // 2, 128), jnp.float32),
                  jax.ShapeDtypeStruct((v // 2, 128), jnp.float32)),
        scratch_types=[
            pltpu.VMEM((64, BLKC), jnp.float32),
            pltpu.VMEM((OROW, 128), jnp.float32),
            pltpu.VMEM((16, 128), jnp.float32),
            pltpu.SemaphoreType.DMA,
        ],
        compiler_params=params,
    )(_t_kernel)
    ut2, mt2 = t_run(u_t, m_t, utail, mtail)

    d_run = functools.partial(
        pl.kernel, mesh=mesh,
        out_type=jax.ShapeDtypeStruct((b,), jnp.float32),
        scratch_types=[
            pltpu.VMEM((bpw,), jnp.int32),
            pltpu.VMEM((bpw,), jnp.int32),
            pltpu.VMEM((bpw,), jnp.int32),
            pltpu.VMEM((bpw,), jnp.int32),
            pltpu.VMEM((bpw,), jnp.float32),
            pltpu.VMEM((RND, 128), jnp.float32),
            pltpu.VMEM((RND, 128), jnp.float32),
            pltpu.VMEM((RND, 128), jnp.float32),
            pltpu.VMEM((RND, 128), jnp.float32),
            pltpu.SemaphoreType.DMA,
            pltpu.SemaphoreType.DMA,
        ],
        compiler_params=params,
    )(functools.partial(_d_kernel, bpw))
    return d_run(ut2, mt2, x[0], x[1])


# speedup vs baseline: 2.1101x; 2.1101x over previous
import functools

import jax
import jax.numpy as jnp
from jax import lax
from jax.experimental import pallas as pl
from jax.experimental.pallas import tpu as pltpu
from jax.experimental.pallas import tpu_sc as plsc

NC = 2
NS = 16
LANES = 16
BLKC = 512
NBLK = 196
LASTC0 = 99456
OROW = BLKC // 2
RND = 128
TAILC = 99968


def _transpose_block(src, dst, j, in_v, out_v, sem, lane):
    c0 = pl.multiple_of(jnp.minimum(j * BLKC, LASTC0), 128)
    copies = [
        pltpu.async_copy(src.at[pl.ds(8 * q, 8), pl.ds(c0, BLKC)],
                         in_v.at[pl.ds(8 * q, 8)], sem)
        for q in range(8)
    ]
    for c in copies:
        c.wait()

    def sub_body(rb, carry):
        rbase = rb * LANES
        cb = 2 * rbase
        for s in range(32):
            coff = (lane + s) & 31
            cvec = cb + coff
            rvec = rbase + jnp.right_shift(coff, 1)
            cbase = (coff & 1) * 64 + lane
            for h in range(4):
                vals = plsc.load_gather(in_v, [lane + h * LANES, cvec])
                plsc.store_scatter(out_v, [rvec, cbase + h * LANES], vals)
        return carry

    lax.fori_loop(0, OROW // LANES, sub_body, 0)
    r0 = pl.multiple_of(jnp.right_shift(c0, 1), 64)
    pltpu.sync_copy(out_v, dst.at[pl.ds(r0, OROW)])


def _t_kernel(u_t, m_t, utail, mtail, ut2, mt2, in_v, out_v, tail_v, sem):
    cid = lax.axis_index("c")
    sid = lax.axis_index("s")
    wid = sid * NC + cid
    lane = lax.iota(jnp.int32, LANES)
    for k in range(13):
        blk = wid + 32 * k

        @pl.when(blk < NBLK)
        def _u():
            _transpose_block(u_t, ut2, blk, in_v, out_v, sem, lane)

        @pl.when(jnp.logical_and(blk >= NBLK, blk < 2 * NBLK))
        def _m():
            _transpose_block(m_t, mt2, blk - NBLK, in_v, out_v, sem, lane)

    @pl.when(wid == 0)
    def _tails():
        pltpu.sync_copy(utail, tail_v)
        pltpu.sync_copy(tail_v, ut2.at[pl.ds(TAILC // 2, 16)])
        pltpu.sync_copy(mtail, tail_v)
        pltpu.sync_copy(tail_v, mt2.at[pl.ds(TAILC // 2, 16)])


def _g_round(t2, row_v, buf, r, sem):
    return pltpu.async_copy(t2.at[row_v.at[pl.ds(r * RND, RND)]], buf, sem)


def _g_compute(uidx_v, midx_v, ubuf, mbuf, o_v, r, lane):
    def group_body(g, carry):
        u16 = uidx_v[pl.ds(r * RND + g * LANES, LANES)]
        m16 = midx_v[pl.ds(r * RND + g * LANES, LANES)]
        uoff = (u16 & 1) * 64
        moff = (m16 & 1) * 64
        rows = lane + g * LANES
        acc = jnp.zeros((LANES,), jnp.float32)
        for dd in range(64):
            ddv = (lane + dd) & 63
            acc = acc + (plsc.load_gather(ubuf, [rows, uoff + ddv])
                         * plsc.load_gather(mbuf, [rows, moff + ddv]))
        o_v[pl.ds(r * RND + g * LANES, LANES)] = acc
        return carry

    lax.fori_loop(0, RND // LANES, group_body, 0)


def _d_kernel(bpw, ut2, mt2, uidx_hbm, midx_hbm, out_hbm,
              uidx_v, midx_v, urow_v, mrow_v, o_v,
              ub0, ub1, mb0, mb1, sem0, sem1):
    cid = lax.axis_index("c")
    sid = lax.axis_index("s")
    wid = sid * NC + cid
    base = wid * bpw
    lane = lax.iota(jnp.int32, LANES)
    nr = bpw // RND
    ubufs, mbufs, sems = (ub0, ub1), (mb0, mb1), (sem0, sem1)

    pltpu.sync_copy(uidx_hbm.at[pl.ds(base, bpw)], uidx_v)
    pltpu.sync_copy(midx_hbm.at[pl.ds(base, bpw)], midx_v)

    for g in range(bpw // LANES):
        sl = pl.ds(g * LANES, LANES)
        urow_v[sl] = jnp.right_shift(uidx_v[sl], 1)
        mrow_v[sl] = jnp.right_shift(midx_v[sl], 1)

    _g_round(ut2, urow_v, ubufs[0], 0, sems[0])
    _g_round(mt2, mrow_v, mbufs[0], 0, sems[0])
    _g_round(ut2, urow_v, ubufs[1], 1, sems[1])
    _g_round(mt2, mrow_v, mbufs[1], 1, sems[1])
    for r in range(nr):
        b = r % 2
        pltpu.make_async_copy(
            ut2.at[urow_v.at[pl.ds(r * RND, RND)]], ubufs[b], sems[b]).wait()
        pltpu.make_async_copy(
            mt2.at[mrow_v.at[pl.ds(r * RND, RND)]], mbufs[b], sems[b]).wait()
        _g_compute(uidx_v, midx_v, ubufs[b], mbufs[b], o_v, r, lane)
        if r + 2 < nr:
            _g_round(ut2, urow_v, ubufs[b], r + 2, sems[b])
            _g_round(mt2, mrow_v, mbufs[b], r + 2, sems[b])

    pltpu.sync_copy(o_v, out_hbm.at[pl.ds(base, bpw)])


def kernel(x, user_emb, movie_emb):
    b = x.shape[1]
    v = user_emb.shape[0]
    bpw = b // (NC * NS)

    u_t = user_emb.T
    m_t = movie_emb.T
    utail = user_emb[TAILC:].reshape(16, 128)
    mtail = movie_emb[TAILC:].reshape(16, 128)
    mesh = plsc.VectorSubcoreMesh(core_axis_name="c", subcore_axis_name="s")
    params = pltpu.CompilerParams(
        use_tc_tiling_on_sc=True, needs_layout_passes=False)

    t_run = functools.partial(
        pl.kernel, mesh=mesh,
        out_type=(jax.ShapeDtypeStruct((v // 2, 128), jnp.float32),
                  jax.ShapeDtypeStruct((v // 2, 128), jnp.float32)),
        scratch_types=[
            pltpu.VMEM((64, BLKC), jnp.float32),
            pltpu.VMEM((OROW, 128), jnp.float32),
            pltpu.VMEM((16, 128), jnp.float32),
            pltpu.SemaphoreType.DMA,
        ],
        compiler_params=params,
    )(_t_kernel)
    ut2, mt2 = t_run(u_t, m_t, utail, mtail)

    d_run = functools.partial(
        pl.kernel, mesh=mesh,
        out_type=jax.ShapeDtypeStruct((b,), jnp.float32),
        scratch_types=[
            pltpu.VMEM((bpw,), jnp.int32),
            pltpu.VMEM((bpw,), jnp.int32),
            pltpu.VMEM((bpw,), jnp.int32),
            pltpu.VMEM((bpw,), jnp.int32),
            pltpu.VMEM((bpw,), jnp.float32),
            pltpu.VMEM((RND, 128), jnp.float32),
            pltpu.VMEM((RND, 128), jnp.float32),
            pltpu.VMEM((RND, 128), jnp.float32),
            pltpu.VMEM((RND, 128), jnp.float32),
            pltpu.SemaphoreType.DMA,
            pltpu.SemaphoreType.DMA,
        ],
        compiler_params=params,
    )(functools.partial(_d_kernel, bpw))
    return d_run(ut2, mt2, x[0], x[1])


# speedup vs baseline: 2.3054x; 1.0925x over previous
import functools

import jax
import jax.numpy as jnp
from jax import lax
from jax.experimental import pallas as pl
from jax.experimental.pallas import tpu as pltpu
from jax.experimental.pallas import tpu_sc as plsc

NC = 2
NS = 16
LANES = 16
BLKC = 512
NBLK = 196
LASTC0 = 99456
OROW = BLKC // 2
RND = 128
TAILC = 99968


def _transpose_block(src, dst, j, in_v, out_v, sem, lane):
    c0 = pl.multiple_of(jnp.minimum(j * BLKC, LASTC0), 128)
    copies = [
        pltpu.async_copy(src.at[pl.ds(8 * q, 8), pl.ds(c0, BLKC)],
                         in_v.at[pl.ds(8 * q, 8)], sem)
        for q in range(8)
    ]
    for c in copies:
        c.wait()

    def sub_body(rb, carry):
        rbase = rb * LANES
        cb = 2 * rbase
        for s in range(32):
            coff = (lane + s) & 31
            cvec = cb + coff
            rvec = rbase + jnp.right_shift(coff, 1)
            cbase = (coff & 1) * 64 + lane
            for h in range(4):
                vals = plsc.load_gather(in_v, [lane + h * LANES, cvec])
                plsc.store_scatter(out_v, [rvec, cbase + h * LANES], vals)
        return carry

    lax.fori_loop(0, OROW // LANES, sub_body, 0)
    r0 = pl.multiple_of(jnp.right_shift(c0, 1), 64)
    pltpu.sync_copy(out_v, dst.at[pl.ds(r0, OROW)])


def _t_kernel(u_t, m_t, utail, mtail, ut2, mt2, in_v, out_v, tail_v, sem):
    cid = lax.axis_index("c")
    sid = lax.axis_index("s")
    wid = sid * NC + cid
    lane = lax.iota(jnp.int32, LANES)
    def blk_body(k, carry):
        blk = wid + 32 * k

        @pl.when(blk < NBLK)
        def _u():
            _transpose_block(u_t, ut2, blk, in_v, out_v, sem, lane)

        @pl.when(jnp.logical_and(blk >= NBLK, blk < 2 * NBLK))
        def _m():
            _transpose_block(m_t, mt2, blk - NBLK, in_v, out_v, sem, lane)

        return carry

    lax.fori_loop(0, 13, blk_body, 0)

    @pl.when(wid == 0)
    def _tails():
        pltpu.sync_copy(utail, tail_v)
        pltpu.sync_copy(tail_v, ut2.at[pl.ds(TAILC // 2, 16)])
        pltpu.sync_copy(mtail, tail_v)
        pltpu.sync_copy(tail_v, mt2.at[pl.ds(TAILC // 2, 16)])


def _g_round(t2, row_v, buf, r, sem):
    return pltpu.async_copy(t2.at[row_v.at[pl.ds(r * RND, RND)]], buf, sem)


def _g_compute(uidx_v, midx_v, ubuf, mbuf, o_v, r, lane):
    def group_body(g, carry):
        u16 = uidx_v[pl.ds(r * RND + g * LANES, LANES)]
        m16 = midx_v[pl.ds(r * RND + g * LANES, LANES)]
        uoff = (u16 & 1) * 64
        moff = (m16 & 1) * 64
        rows = lane + g * LANES
        acc = jnp.zeros((LANES,), jnp.float32)
        for dd in range(64):
            ddv = (lane + dd) & 63
            acc = acc + (plsc.load_gather(ubuf, [rows, uoff + ddv])
                         * plsc.load_gather(mbuf, [rows, moff + ddv]))
        o_v[pl.ds(r * RND + g * LANES, LANES)] = acc
        return carry

    lax.fori_loop(0, RND // LANES, group_body, 0)


def _d_kernel(bpw, ut2, mt2, uidx_hbm, midx_hbm, out_hbm,
              uidx_v, midx_v, urow_v, mrow_v, o_v,
              ub0, ub1, mb0, mb1, sem0, sem1):
    cid = lax.axis_index("c")
    sid = lax.axis_index("s")
    wid = sid * NC + cid
    base = wid * bpw
    lane = lax.iota(jnp.int32, LANES)
    nr = bpw // RND
    ubufs, mbufs, sems = (ub0, ub1), (mb0, mb1), (sem0, sem1)

    pltpu.sync_copy(uidx_hbm.at[pl.ds(base, bpw)], uidx_v)
    pltpu.sync_copy(midx_hbm.at[pl.ds(base, bpw)], midx_v)

    for g in range(bpw // LANES):
        sl = pl.ds(g * LANES, LANES)
        urow_v[sl] = jnp.right_shift(uidx_v[sl], 1)
        mrow_v[sl] = jnp.right_shift(midx_v[sl], 1)

    _g_round(ut2, urow_v, ubufs[0], 0, sems[0])
    _g_round(mt2, mrow_v, mbufs[0], 0, sems[0])
    _g_round(ut2, urow_v, ubufs[1], 1, sems[1])
    _g_round(mt2, mrow_v, mbufs[1], 1, sems[1])
    for r in range(nr):
        b = r % 2
        pltpu.make_async_copy(
            ut2.at[urow_v.at[pl.ds(r * RND, RND)]], ubufs[b], sems[b]).wait()
        pltpu.make_async_copy(
            mt2.at[mrow_v.at[pl.ds(r * RND, RND)]], mbufs[b], sems[b]).wait()
        _g_compute(uidx_v, midx_v, ubufs[b], mbufs[b], o_v, r, lane)
        if r + 2 < nr:
            _g_round(ut2, urow_v, ubufs[b], r + 2, sems[b])
            _g_round(mt2, mrow_v, mbufs[b], r + 2, sems[b])

    pltpu.sync_copy(o_v, out_hbm.at[pl.ds(base, bpw)])


def kernel(x, user_emb, movie_emb):
    b = x.shape[1]
    v = user_emb.shape[0]
    bpw = b // (NC * NS)

    u_t = user_emb.T
    m_t = movie_emb.T
    utail = user_emb[TAILC:].reshape(16, 128)
    mtail = movie_emb[TAILC:].reshape(16, 128)
    mesh = plsc.VectorSubcoreMesh(core_axis_name="c", subcore_axis_name="s")
    params = pltpu.CompilerParams(
        use_tc_tiling_on_sc=True, needs_layout_passes=False)

    t_run = functools.partial(
        pl.kernel, mesh=mesh,
        out_type=(jax.ShapeDtypeStruct((v // 2, 128), jnp.float32),
                  jax.ShapeDtypeStruct((v // 2, 128), jnp.float32)),
        scratch_types=[
            pltpu.VMEM((64, BLKC), jnp.float32),
            pltpu.VMEM((OROW, 128), jnp.float32),
            pltpu.VMEM((16, 128), jnp.float32),
            pltpu.SemaphoreType.DMA,
        ],
        compiler_params=params,
    )(_t_kernel)
    ut2, mt2 = t_run(u_t, m_t, utail, mtail)

    d_run = functools.partial(
        pl.kernel, mesh=mesh,
        out_type=jax.ShapeDtypeStruct((b,), jnp.float32),
        scratch_types=[
            pltpu.VMEM((bpw,), jnp.int32),
            pltpu.VMEM((bpw,), jnp.int32),
            pltpu.VMEM((bpw,), jnp.int32),
            pltpu.VMEM((bpw,), jnp.int32),
            pltpu.VMEM((bpw,), jnp.float32),
            pltpu.VMEM((RND, 128), jnp.float32),
            pltpu.VMEM((RND, 128), jnp.float32),
            pltpu.VMEM((RND, 128), jnp.float32),
            pltpu.VMEM((RND, 128), jnp.float32),
            pltpu.SemaphoreType.DMA,
            pltpu.SemaphoreType.DMA,
        ],
        compiler_params=params,
    )(functools.partial(_d_kernel, bpw))
    return d_run(ut2, mt2, x[0], x[1])


# speedup vs baseline: 2.7160x; 1.1781x over previous
import functools

import jax
import jax.numpy as jnp
from jax import lax
from jax.experimental import pallas as pl
from jax.experimental.pallas import tpu as pltpu
from jax.experimental.pallas import tpu_sc as plsc

NC = 2
NS = 16
LANES = 16
BLKC = 512
NBLK = 196
LASTC0 = 99456
OROW = BLKC // 2
RND = 128
TAILC = 99968


def _blk_c0(j):
    return pl.multiple_of(jnp.minimum(j * BLKC, LASTC0), 128)


def _stage_block(src, j, in_v, sem):
    c0 = _blk_c0(j)
    for q in range(8):
        pltpu.async_copy(src.at[pl.ds(8 * q, 8), pl.ds(c0, BLKC)],
                         in_v.at[pl.ds(8 * q, 8)], sem)


def _wait_block(src, j, in_v, sem):
    c0 = _blk_c0(j)
    for q in range(8):
        pltpu.make_async_copy(src.at[pl.ds(8 * q, 8), pl.ds(c0, BLKC)],
                              in_v.at[pl.ds(8 * q, 8)], sem).wait()


def _transpose_block(src, dst, j, in_v, out_v, sem, lane):
    c0 = _blk_c0(j)

    def sub_body(rb, carry):
        rbase = rb * LANES
        cb = 2 * rbase
        for s in range(32):
            coff = (lane + s) & 31
            cvec = cb + coff
            rvec = rbase + jnp.right_shift(coff, 1)
            cbase = (coff & 1) * 64 + lane
            for h in range(4):
                vals = plsc.load_gather(in_v, [lane + h * LANES, cvec])
                plsc.store_scatter(out_v, [rvec, cbase + h * LANES], vals)
        return carry

    lax.fori_loop(0, OROW // LANES, sub_body, 0)
    r0 = pl.multiple_of(jnp.right_shift(c0, 1), 64)
    pltpu.sync_copy(out_v, dst.at[pl.ds(r0, OROW)])


def _stage_any(u_t, m_t, blk, buf, sem):
    @pl.when(blk < NBLK)
    def _u():
        _stage_block(u_t, blk, buf, sem)

    @pl.when(jnp.logical_and(blk >= NBLK, blk < 2 * NBLK))
    def _m():
        _stage_block(m_t, blk - NBLK, buf, sem)


def _wait_any(u_t, m_t, blk, buf, sem):
    @pl.when(blk < NBLK)
    def _u():
        _wait_block(u_t, blk, buf, sem)

    @pl.when(jnp.logical_and(blk >= NBLK, blk < 2 * NBLK))
    def _m():
        _wait_block(m_t, blk - NBLK, buf, sem)


def _consume_any(u_t, m_t, ut2, mt2, blk, buf, out_v, sem, lane):
    @pl.when(blk < NBLK)
    def _u():
        _transpose_block(u_t, ut2, blk, buf, out_v, sem, lane)

    @pl.when(jnp.logical_and(blk >= NBLK, blk < 2 * NBLK))
    def _m():
        _transpose_block(m_t, mt2, blk - NBLK, buf, out_v, sem, lane)


def _t_kernel(u_t, m_t, utail, mtail, ut2, mt2,
              in0, in1, out_v, tail_v, semA, semB):
    cid = lax.axis_index("c")
    sid = lax.axis_index("s")
    wid = sid * NC + cid
    lane = lax.iota(jnp.int32, LANES)

    _stage_any(u_t, m_t, wid, in0, semA)
    _stage_any(u_t, m_t, wid + 32, in1, semB)

    def blk_body(i, carry):
        b0 = wid + 64 * i
        b1 = b0 + 32
        _wait_any(u_t, m_t, b0, in0, semA)
        _consume_any(u_t, m_t, ut2, mt2, b0, in0, out_v, semA, lane)
        _stage_any(u_t, m_t, b0 + 64, in0, semA)
        _wait_any(u_t, m_t, b1, in1, semB)
        _consume_any(u_t, m_t, ut2, mt2, b1, in1, out_v, semB, lane)
        _stage_any(u_t, m_t, b1 + 64, in1, semB)
        return carry

    lax.fori_loop(0, 7, blk_body, 0)

    @pl.when(wid == 0)
    def _tails():
        pltpu.sync_copy(utail, tail_v)
        pltpu.sync_copy(tail_v, ut2.at[pl.ds(TAILC // 2, 16)])
        pltpu.sync_copy(mtail, tail_v)
        pltpu.sync_copy(tail_v, mt2.at[pl.ds(TAILC // 2, 16)])


def _g_round(t2, row_v, buf, r, sem):
    return pltpu.async_copy(t2.at[row_v.at[pl.ds(r * RND, RND)]], buf, sem)


def _g_compute(uidx_v, midx_v, ubuf, mbuf, o_v, r, lane):
    def group_body(g, carry):
        u16 = uidx_v[pl.ds(r * RND + g * LANES, LANES)]
        m16 = midx_v[pl.ds(r * RND + g * LANES, LANES)]
        uoff = (u16 & 1) * 64
        moff = (m16 & 1) * 64
        rows = lane + g * LANES
        acc = jnp.zeros((LANES,), jnp.float32)
        for dd in range(64):
            ddv = (lane + dd) & 63
            acc = acc + (plsc.load_gather(ubuf, [rows, uoff + ddv])
                         * plsc.load_gather(mbuf, [rows, moff + ddv]))
        o_v[pl.ds(r * RND + g * LANES, LANES)] = acc
        return carry

    lax.fori_loop(0, RND // LANES, group_body, 0)


def _d_kernel(bpw, ut2, mt2, uidx_hbm, midx_hbm, out_hbm,
              uidx_v, midx_v, urow_v, mrow_v, o_v,
              ub0, ub1, mb0, mb1, sem0, sem1):
    cid = lax.axis_index("c")
    sid = lax.axis_index("s")
    wid = sid * NC + cid
    base = wid * bpw
    lane = lax.iota(jnp.int32, LANES)
    nr = bpw // RND
    ubufs, mbufs, sems = (ub0, ub1), (mb0, mb1), (sem0, sem1)

    pltpu.sync_copy(uidx_hbm.at[pl.ds(base, bpw)], uidx_v)
    pltpu.sync_copy(midx_hbm.at[pl.ds(base, bpw)], midx_v)

    for g in range(bpw // LANES):
        sl = pl.ds(g * LANES, LANES)
        urow_v[sl] = jnp.right_shift(uidx_v[sl], 1)
        mrow_v[sl] = jnp.right_shift(midx_v[sl], 1)

    _g_round(ut2, urow_v, ubufs[0], 0, sems[0])
    _g_round(mt2, mrow_v, mbufs[0], 0, sems[0])
    _g_round(ut2, urow_v, ubufs[1], 1, sems[1])
    _g_round(mt2, mrow_v, mbufs[1], 1, sems[1])
    for r in range(nr):
        b = r % 2
        pltpu.make_async_copy(
            ut2.at[urow_v.at[pl.ds(r * RND, RND)]], ubufs[b], sems[b]).wait()
        pltpu.make_async_copy(
            mt2.at[mrow_v.at[pl.ds(r * RND, RND)]], mbufs[b], sems[b]).wait()
        _g_compute(uidx_v, midx_v, ubufs[b], mbufs[b], o_v, r, lane)
        if r + 2 < nr:
            _g_round(ut2, urow_v, ubufs[b], r + 2, sems[b])
            _g_round(mt2, mrow_v, mbufs[b], r + 2, sems[b])

    pltpu.sync_copy(o_v, out_hbm.at[pl.ds(base, bpw)])


def kernel(x, user_emb, movie_emb):
    b = x.shape[1]
    v = user_emb.shape[0]
    bpw = b // (NC * NS)

    u_t = user_emb.T
    m_t = movie_emb.T
    utail = user_emb[TAILC:].reshape(16, 128)
    mtail = movie_emb[TAILC:].reshape(16, 128)
    mesh = plsc.VectorSubcoreMesh(core_axis_name="c", subcore_axis_name="s")
    params = pltpu.CompilerParams(
        use_tc_tiling_on_sc=True, needs_layout_passes=False)

    t_run = functools.partial(
        pl.kernel, mesh=mesh,
        out_type=(jax.ShapeDtypeStruct((v // 2, 128), jnp.float32),
                  jax.ShapeDtypeStruct((v // 2, 128), jnp.float32)),
        scratch_types=[
            pltpu.VMEM((64, BLKC), jnp.float32),
            pltpu.VMEM((64, BLKC), jnp.float32),
            pltpu.VMEM((OROW, 128), jnp.float32),
            pltpu.VMEM((16, 128), jnp.float32),
            pltpu.SemaphoreType.DMA,
            pltpu.SemaphoreType.DMA,
        ],
        compiler_params=params,
    )(_t_kernel)
    ut2, mt2 = t_run(u_t, m_t, utail, mtail)

    d_run = functools.partial(
        pl.kernel, mesh=mesh,
        out_type=jax.ShapeDtypeStruct((b,), jnp.float32),
        scratch_types=[
            pltpu.VMEM((bpw,), jnp.int32),
            pltpu.VMEM((bpw,), jnp.int32),
            pltpu.VMEM((bpw,), jnp.int32),
            pltpu.VMEM((bpw,), jnp.int32),
            pltpu.VMEM((bpw,), jnp.float32),
            pltpu.VMEM((RND, 128), jnp.float32),
            pltpu.VMEM((RND, 128), jnp.float32),
            pltpu.VMEM((RND, 128), jnp.float32),
            pltpu.VMEM((RND, 128), jnp.float32),
            pltpu.SemaphoreType.DMA,
            pltpu.SemaphoreType.DMA,
        ],
        compiler_params=params,
    )(functools.partial(_d_kernel, bpw))
    return d_run(ut2, mt2, x[0], x[1])
